# Initial kernel scaffold; baseline (speedup 1.0000x reference)
import jax
import jax.numpy as jnp
from jax.experimental import pallas as pl


def kernel(queries, keys, top_k):
    raise NotImplementedError("write your pallas kernel here")


# speedup vs baseline: 1.3729x; 1.3729x over previous
import functools

import jax
import jax.numpy as jnp
from jax.experimental import pallas as pl
from jax.experimental.pallas import tpu as pltpu

_TOPK = 8
_NEG = float("-inf")
_BIGI = 2**30


def _extract_topk(s, idx, n):
    q, w = s.shape
    pos_iota = jax.lax.broadcasted_iota(jnp.int32, (q, w), 1)
    vals, idxs = [], []
    for _ in range(n):
        m = jnp.max(s, axis=1, keepdims=True)
        pos = jnp.min(jnp.where(s == m, pos_iota, _BIGI), axis=1, keepdims=True)
        hit = pos_iota == pos
        vals.append(m)
        idxs.append(jnp.sum(jnp.where(hit, idx, 0), axis=1, keepdims=True))
        s = jnp.where(hit, _NEG, s)
    return jnp.concatenate(vals, axis=1), jnp.concatenate(idxs, axis=1)


def _topk_kernel(q_ref, k_ref, vals_ref, idx_ref, qn_ref, rv_ref, ri_ref,
                 *, block, topk, kvalid):
    i = pl.program_id(0)
    nb = pl.num_programs(0)
    qdim = q_ref.shape[0]

    @pl.when(i == 0)
    def _init():
        q = q_ref[...]
        qn2 = jnp.sum(q * q, axis=1, keepdims=True)
        qn_ref[...] = q / jnp.maximum(jnp.sqrt(qn2), 1e-12)
        rv_ref[...] = jnp.full((qdim, topk), _NEG, jnp.float32)
        ri_ref[...] = jnp.zeros((qdim, topk), jnp.int32)

    kb = k_ref[...]
    kn2 = jnp.sum(kb * kb, axis=1, keepdims=True)
    kn = kb / jnp.maximum(jnp.sqrt(kn2), 1e-12)
    s = jax.lax.dot_general(qn_ref[...], kn, (((1,), (1,)), ((), ())),
                            preferred_element_type=jnp.float32)

    gidx = i * block + jax.lax.broadcasted_iota(jnp.int32, (qdim, block), 1)
    if kvalid % block != 0:
        s = jnp.where(gidx < kvalid, s, _NEG)
    bv, bi = _extract_topk(s, gidx, topk)

    cv = jnp.concatenate([rv_ref[...], bv], axis=1)
    ci = jnp.concatenate([ri_ref[...], bi], axis=1)
    nv, ni = _extract_topk(cv, ci, topk)
    rv_ref[...] = nv
    ri_ref[...] = ni

    @pl.when(i == nb - 1)
    def _emit():
        vals_ref[...] = rv_ref[...]
        idx_ref[...] = ri_ref[...]


def _pick_block(k):
    for b in (2000, 2048, 1600, 1024, 1000, 800, 512, 400, 256, 200, 128, 8):
        if k % b == 0 and b % 8 == 0:
            return b
    return None


@functools.partial(jax.jit, static_argnums=())
def kernel(queries, keys, top_k):
    del top_k
    qdim, d = queries.shape
    k = keys.shape[0]
    block = _pick_block(k)
    kvalid = k
    if block is None:
        block = 2048
        pad = (-k) % block
        keys = jnp.pad(keys, ((0, pad), (0, 0)))
        k = k + pad
    nb = k // block

    body = functools.partial(_topk_kernel, block=block, topk=_TOPK,
                             kvalid=kvalid)
    vals, idx = pl.pallas_call(
        body,
        grid=(nb,),
        in_specs=[
            pl.BlockSpec((qdim, d), lambda i: (0, 0)),
            pl.BlockSpec((block, d), lambda i: (i, 0)),
        ],
        out_specs=[
            pl.BlockSpec((qdim, _TOPK), lambda i: (0, 0)),
            pl.BlockSpec((qdim, _TOPK), lambda i: (0, 0)),
        ],
        out_shape=[
            jax.ShapeDtypeStruct((qdim, _TOPK), jnp.float32),
            jax.ShapeDtypeStruct((qdim, _TOPK), jnp.int32),
        ],
        scratch_shapes=[
            pltpu.VMEM((qdim, d), jnp.float32),
            pltpu.VMEM((qdim, _TOPK), jnp.float32),
            pltpu.VMEM((qdim, _TOPK), jnp.int32),
        ],
    )(queries, keys)
    return vals, idx


# speedup vs baseline: 3.6374x; 2.6495x over previous
import functools

import jax
import jax.numpy as jnp
from jax.experimental import pallas as pl
from jax.experimental.pallas import tpu as pltpu

_TOPK = 8
_NEG = float("-inf")
_BIGI = 2**30
_LANES = 128


def _extract_topk(s, idx, n):
    q, w = s.shape
    pos_iota = jax.lax.broadcasted_iota(jnp.int32, (q, w), 1)
    vals, idxs = [], []
    for _ in range(n):
        m = jnp.max(s, axis=1, keepdims=True)
        pos = jnp.min(jnp.where(s == m, pos_iota, _BIGI), axis=1, keepdims=True)
        hit = pos_iota == pos
        vals.append(m)
        if idx is None:
            idxs.append(pos)
        else:
            idxs.append(jnp.sum(jnp.where(hit, idx, 0), axis=1, keepdims=True))
        s = jnp.where(hit, _NEG, s)
    return jnp.concatenate(vals, axis=1), jnp.concatenate(idxs, axis=1)


def _lane_sweep(s, base):
    q, b = s.shape
    cfull, rem = divmod(b, _LANES)
    lane = jax.lax.broadcasted_iota(jnp.int32, (q, _LANES), 1)
    neg = jnp.full((q, _LANES), _NEG, jnp.float32)
    zero = jnp.zeros((q, _LANES), jnp.int32)
    m1, m2, m3, m4 = neg, neg, neg, neg
    a1, a2, a3, a4 = zero, zero, zero, zero
    ncols = cfull + (1 if rem else 0)
    for j in range(ncols):
        if j < cfull:
            col = s[:, j * _LANES:(j + 1) * _LANES]
        else:
            col = jnp.concatenate(
                [s[:, cfull * _LANES:],
                 jnp.full((q, _LANES - rem), _NEG, jnp.float32)], axis=1)
        g1 = col > m1
        g2 = col > m2
        g3 = col > m3
        g4 = col > m4
        m4 = jnp.where(g3, m3, jnp.where(g4, col, m4))
        a4 = jnp.where(g3, a3, jnp.where(g4, j, a4))
        m3 = jnp.where(g2, m2, jnp.where(g3, col, m3))
        a3 = jnp.where(g2, a2, jnp.where(g3, j, a3))
        m2 = jnp.where(g1, m1, jnp.where(g2, col, m2))
        a2 = jnp.where(g1, a1, jnp.where(g2, j, a2))
        m1 = jnp.where(g1, col, m1)
        a1 = jnp.where(g1, j, a1)
    idx = [base + a * _LANES + lane for a in (a1, a2, a3, a4)]
    return (m1, m2, m3, m4), idx


def _merge_rounds(rv, ri, ms, idxs, topk):
    q = rv.shape[0]
    wide = topk + _LANES
    negcol = jnp.full((q, topk), _NEG, jnp.float32)
    sent = -(1 + topk) - jax.lax.broadcasted_iota(jnp.int32, (q, wide), 1)
    sentcol = -(1 + topk) - _LANES - jax.lax.broadcasted_iota(
        jnp.int32, (q, topk), 1)
    mc = jnp.concatenate([rv, ms[0]], axis=1)
    ic = jnp.concatenate([ri, idxs[0]], axis=1)
    n1v = jnp.concatenate([negcol, ms[1]], axis=1)
    n1i = jnp.concatenate([sentcol, idxs[1]], axis=1)
    n2v = jnp.concatenate([negcol, ms[2]], axis=1)
    n2i = jnp.concatenate([sentcol, idxs[2]], axis=1)
    n3v = jnp.concatenate([negcol, ms[3]], axis=1)
    n3i = jnp.concatenate([sentcol, idxs[3]], axis=1)
    pops = jnp.zeros((q, wide), jnp.int32)
    vals, out_idx = [], []
    for _ in range(topk):
        m = jnp.max(mc, axis=1, keepdims=True)
        pos = jnp.min(jnp.where(mc == m, ic, _BIGI), axis=1, keepdims=True)
        vals.append(m)
        out_idx.append(pos)
        hit = ic == pos
        mc = jnp.where(hit, n1v, mc)
        ic = jnp.where(hit, n1i, ic)
        n1v = jnp.where(hit, n2v, n1v)
        n1i = jnp.where(hit, n2i, n1i)
        n2v = jnp.where(hit, n3v, n2v)
        n2i = jnp.where(hit, n3i, n2i)
        n3v = jnp.where(hit, _NEG, n3v)
        n3i = jnp.where(hit, sent, n3i)
        pops = pops + hit.astype(jnp.int32)
    return (jnp.concatenate(vals, axis=1), jnp.concatenate(out_idx, axis=1),
            jnp.max(pops))


def _topk_kernel(q_ref, k_ref, vals_ref, idx_ref, qn_ref, rv_ref, ri_ref,
                 *, block, topk, kvalid):
    i = pl.program_id(0)
    nb = pl.num_programs(0)
    qdim = q_ref.shape[0]

    @pl.when(i == 0)
    def _init():
        q = q_ref[...]
        qn2 = jnp.sum(q * q, axis=1, keepdims=True)
        qn_ref[...] = q / jnp.maximum(jnp.sqrt(qn2), 1e-12)
        rv_ref[...] = jnp.full((qdim, topk), _NEG, jnp.float32)
        ri_ref[...] = -1 - jax.lax.broadcasted_iota(jnp.int32, (qdim, topk), 1)

    kb = k_ref[...]
    kn2 = jnp.sum(kb * kb, axis=1, keepdims=True)
    kn = kb / jnp.maximum(jnp.sqrt(kn2), 1e-12)
    s = jax.lax.dot_general(qn_ref[...], kn, (((1,), (1,)), ((), ())),
                            preferred_element_type=jnp.float32)

    base = i * block
    if kvalid % block != 0:
        gidx = base + jax.lax.broadcasted_iota(jnp.int32, s.shape, 1)
        s = jnp.where(gidx < kvalid, s, _NEG)
    rv, ri = rv_ref[...], ri_ref[...]
    ms, idxs = _lane_sweep(s, base)
    cv, ci, maxpops = _merge_rounds(rv, ri, ms, idxs, topk)

    def _fallback():
        bv, bpos = _extract_topk(s, None, topk)
        bi = base + bpos
        return _extract_topk(jnp.concatenate([rv, bv], axis=1),
                             jnp.concatenate([ri, bi], axis=1), topk)

    nv, ni = jax.lax.cond(maxpops >= 4, _fallback, lambda: (cv, ci))
    rv_ref[...] = nv
    ri_ref[...] = ni

    @pl.when(i == nb - 1)
    def _emit():
        vals_ref[...] = rv_ref[...]
        idx_ref[...] = ri_ref[...]


def _pick_block(k):
    for b in (4000, 4096, 2048, 2000, 1600, 1024, 1000, 800, 512, 400, 256,
              200, 128, 8):
        if k % b == 0 and b % 8 == 0:
            return b
    return None


def kernel(queries, keys, top_k):
    del top_k
    qdim, d = queries.shape
    k = keys.shape[0]
    block = _pick_block(k)
    kvalid = k
    if block is None:
        block = 4096
        pad = (-k) % block
        keys = jnp.pad(keys, ((0, pad), (0, 0)), constant_values=0.0)
        k = k + pad
    nb = k // block

    body = functools.partial(_topk_kernel, block=block, topk=_TOPK,
                             kvalid=kvalid)
    vals, idx = pl.pallas_call(
        body,
        grid=(nb,),
        in_specs=[
            pl.BlockSpec((qdim, d), lambda i: (0, 0)),
            pl.BlockSpec((block, d), lambda i: (i, 0)),
        ],
        out_specs=[
            pl.BlockSpec((qdim, _TOPK), lambda i: (0, 0)),
            pl.BlockSpec((qdim, _TOPK), lambda i: (0, 0)),
        ],
        out_shape=[
            jax.ShapeDtypeStruct((qdim, _TOPK), jnp.float32),
            jax.ShapeDtypeStruct((qdim, _TOPK), jnp.int32),
        ],
        scratch_shapes=[
            pltpu.VMEM((qdim, d), jnp.float32),
            pltpu.VMEM((qdim, _TOPK), jnp.float32),
            pltpu.VMEM((qdim, _TOPK), jnp.int32),
        ],
        compiler_params=pltpu.CompilerParams(
            vmem_limit_bytes=64 * 1024 * 1024),
    )(queries, keys)
    return vals, idx
